# Initial kernel scaffold; baseline (speedup 1.0000x reference)
import jax
import jax.numpy as jnp
from jax.experimental import pallas as pl


def kernel(h_lig, h_rec, x_lig, x_rec, edge_index_ll, edge_index_rl, params):
    raise NotImplementedError("write your pallas kernel here")


# speedup vs baseline: 1.0042x; 1.0042x over previous
import functools

import jax
import jax.numpy as jnp
from jax.experimental import pallas as pl


_BE = 640
_BN = 1000


def _silu(x):
    return x * jax.nn.sigmoid(x)


def _edge_body(hs_ref, hd_ref, xd_ref,
               w1a, w1b, w1c, b1, w2, b2,
               c1a, c1b, c1c, bc1, c2r, bc2r,
               msgh_ref, msgx_ref):
    hs = hs_ref[...]
    hd = hd_ref[...]
    xd = xd_ref[...]
    d2 = jnp.sum(xd * xd, axis=1, keepdims=True) + 1e-12
    dij = jnp.sqrt(d2)
    xdn = xd / (dij + 1e-9)
    pre = (jnp.dot(hs, w1a[...], preferred_element_type=jnp.float32)
           + jnp.dot(hd, w1b[...], preferred_element_type=jnp.float32)
           + dij * w1c[...] + b1[...])
    y = _silu(pre)
    msgh_ref[...] = _silu(jnp.dot(y, w2[...], preferred_element_type=jnp.float32)
                          + b2[...])
    prec = (jnp.dot(hs, c1a[...], preferred_element_type=jnp.float32)
            + jnp.dot(hd, c1b[...], preferred_element_type=jnp.float32)
            + dij * c1c[...] + bc1[...])
    yc = _silu(prec)
    scale = _silu(jnp.dot(yc, c2r[...], preferred_element_type=jnp.float32)
                  + bc2r[...])
    msgx_ref[...] = scale * xdn


@functools.partial(jax.jit, static_argnames=())
def _edge_compute(hs, hd, xd8, ew):
    e = hs.shape[0]
    grid = (e // _BE,)
    blk = lambda w: pl.BlockSpec((_BE, w), lambda i: (i, 0))
    full = lambda a: pl.BlockSpec(a.shape, lambda i: (0, 0))
    in_specs = [blk(128), blk(128), blk(8)] + [full(w) for w in ew]
    out_specs = [blk(128), blk(8)]
    return pl.pallas_call(
        _edge_body,
        grid=grid,
        in_specs=in_specs,
        out_specs=out_specs,
        out_shape=[jax.ShapeDtypeStruct((e, 128), jnp.float32),
                   jax.ShapeDtypeStruct((e, 8), jnp.float32)],
    )(hs, hd, xd8, *ew)


def _node_body(h_ref, hn_ref, n1a, n1b, nb1, n2, nb2, out_ref):
    h = h_ref[...]
    hn = hn_ref[...]
    y = _silu(jnp.dot(h, n1a[...], preferred_element_type=jnp.float32)
              + jnp.dot(hn, n1b[...], preferred_element_type=jnp.float32)
              + nb1[...])
    out_ref[...] = h + jnp.dot(y, n2[...], preferred_element_type=jnp.float32) + nb2[...]


def _node_compute(h, hn, nw):
    n = h.shape[0]
    grid = (n // _BN,)
    blk = pl.BlockSpec((_BN, 128), lambda i: (i, 0))
    full = lambda a: pl.BlockSpec(a.shape, lambda i: (0, 0))
    return pl.pallas_call(
        _node_body,
        grid=grid,
        in_specs=[blk, blk] + [full(w) for w in nw],
        out_specs=blk,
        out_shape=jax.ShapeDtypeStruct((n, 128), jnp.float32),
    )(h, hn, *nw)


def _split_edge_weights(lp, et, in_s):
    w1 = lp['edge_' + et + '_1']['W']
    b1 = lp['edge_' + et + '_1']['b']
    w2 = lp['edge_' + et + '_2']['W']
    b2 = lp['edge_' + et + '_2']['b']
    c1 = lp['coord_' + et + '_1']['W']
    bc1 = lp['coord_' + et + '_1']['b']
    c2 = lp['coord_' + et + '_2']['W']
    bc2 = lp['coord_' + et + '_2']['b']
    c2r = jnp.tile(c2, (1, 8))
    bc2r = jnp.tile(bc2[None, :], (1, 8))
    return (w1[:in_s], w1[in_s:2 * in_s], w1[2 * in_s:2 * in_s + 1], b1[None, :],
            w2, b2[None, :],
            c1[:in_s], c1[in_s:2 * in_s], c1[2 * in_s:2 * in_s + 1], bc1[None, :],
            c2r, bc2r)


def kernel(h_lig, h_rec, x_lig, x_rec, edge_index_ll, edge_index_rl, params):
    n_lig = h_lig.shape[0]
    hl, xl = h_lig, x_lig
    src_ll, dst_ll = edge_index_ll[0], edge_index_ll[1]
    src_rl, dst_rl = edge_index_rl[0], edge_index_rl[1]
    for li, lp in enumerate(params):
        in_s = hl.shape[1]
        h_neigh = jnp.zeros((n_lig, 128), jnp.float32)
        x_neigh8 = jnp.zeros((n_lig, 8), jnp.float32)
        for et, h_src_all, x_src_all, src, dst in (
                ('ll', hl, xl, src_ll, dst_ll),
                ('rl', h_rec, x_rec, src_rl, dst_rl)):
            hs = jnp.take(h_src_all, src, axis=0)
            hd = jnp.take(hl, dst, axis=0)
            xd = jnp.take(x_src_all, src, axis=0) - jnp.take(xl, dst, axis=0)
            xd8 = jnp.pad(xd, ((0, 0), (0, 5)))
            ew = _split_edge_weights(lp, et, in_s)
            msg_h, msg_x8 = _edge_compute(hs, hd, xd8, ew)
            h_neigh = h_neigh.at[dst].add(msg_h)
            x_neigh8 = x_neigh8.at[dst].add(msg_x8)
        n1 = lp['node_1']['W']
        nw = (n1[:in_s], n1[in_s:], lp['node_1']['b'][None, :],
              lp['node_2']['W'], lp['node_2']['b'][None, :])
        hl = _node_compute(hl, h_neigh, nw)
        xl = xl + x_neigh8[:, :3]
    return (hl, h_rec, xl, x_rec)


# speedup vs baseline: 1.1662x; 1.1614x over previous
import functools

import jax
import jax.numpy as jnp
from jax import lax
from jax.experimental import pallas as pl
from jax.experimental.pallas import tpu as pltpu
from jax.experimental.pallas import tpu_sc as plsc

N_NODE = 10000
N_EDGE = 160000
NC = 2
NS = 16
NW = NC * NS
C = 128
NCH = 40
E_PAD = NW * NCH * C
N_TAB = N_NODE + 8
ACC_N = 10240
ROWS_T = ACC_N // NS
XACC_N = 1280
XROWS_T = XACC_N // NS
DPAD = N_NODE
HALF = 5120
HACC_N = 5248
HROWS_T = HACC_N // NS

_BE = 640
_BN = 1000


def _silu(x):
    return x * jax.nn.sigmoid(x)


def _gather_stream(tab, idx_v, out, bufs, sems, wid):
    gsA, gsB, wsA, wsB = sems

    def g_mk(g, slot, sem):
        return pltpu.make_async_copy(tab.at[idx_v.at[g]], bufs.at[slot], sem)

    def w_mk(g, slot, sem):
        off = pl.multiple_of((wid * NCH + g) * C, 8)
        return pltpu.make_async_copy(bufs.at[slot], out.at[pl.ds(off, C)], sem)

    def g_issue(g, slot, sem):
        pltpu.async_copy(tab.at[idx_v.at[g]], bufs.at[slot], sem)

    def w_issue(g, slot, sem):
        off = pl.multiple_of((wid * NCH + g) * C, 8)
        pltpu.async_copy(bufs.at[slot], out.at[pl.ds(off, C)], sem)

    g_issue(0, 0, gsA)
    g_issue(1, 1, gsA)

    def body(i, carry):
        g0 = i * 4
        g_mk(g0, 0, gsA).wait()
        g_mk(g0 + 1, 1, gsA).wait()

        @pl.when(g0 > 0)
        def _():
            w_mk(g0 - 2, 2, wsB).wait()
            w_mk(g0 - 1, 3, wsB).wait()

        g_issue(g0 + 2, 2, gsB)
        g_issue(g0 + 3, 3, gsB)
        w_issue(g0, 0, wsA)
        w_issue(g0 + 1, 1, wsA)
        g_mk(g0 + 2, 2, gsB).wait()
        g_mk(g0 + 3, 3, gsB).wait()
        w_mk(g0, 0, wsA).wait()
        w_mk(g0 + 1, 1, wsA).wait()

        @pl.when(g0 + 4 < NCH)
        def _():
            g_issue(g0 + 4, 0, gsA)
            g_issue(g0 + 5, 1, gsA)

        w_issue(g0 + 2, 2, wsB)
        w_issue(g0 + 3, 3, wsB)
        return carry

    lax.fori_loop(0, NCH // 4, body, 0)
    w_mk(NCH - 2, 2, wsB).wait()
    w_mk(NCH - 1, 3, wsB).wait()


def _sc_gather_layer(hl_t, hr_t, xl_t, xr_t,
                     src_ll2, dst_ll2, src_rl2, dst_rl2):
    mesh = plsc.VectorSubcoreMesh(core_axis_name="c", subcore_axis_name="s")
    eshape = jax.ShapeDtypeStruct((E_PAD, 128), jnp.float32)

    @functools.partial(
        pl.kernel, mesh=mesh,
        out_type=[eshape] * 8,
        scratch_types=[
            pltpu.VMEM((NCH, C), jnp.int32),
            pltpu.VMEM((NCH, C), jnp.int32),
            pltpu.VMEM((NCH, C), jnp.int32),
            pltpu.VMEM((NCH, C), jnp.int32),
            pltpu.VMEM((4, C, 128), jnp.float32),
        ] + [pltpu.SemaphoreType.DMA] * 4,
    )
    def k(hl, hr, xl, xr, ill_s, ill_d, irl_s, irl_d,
          o_hs_ll, o_hd_ll, o_xs_ll, o_xl_ll,
          o_hs_rl, o_hd_rl, o_xs_rl, o_xl_rl,
          iv_sll, iv_dll, iv_srl, iv_drl, bufs,
          s0, s1, s2, s3):
        wid = lax.axis_index("s") * NC + lax.axis_index("c")
        row0 = wid * NCH
        pltpu.sync_copy(ill_s.at[pl.ds(row0, NCH)], iv_sll)
        pltpu.sync_copy(ill_d.at[pl.ds(row0, NCH)], iv_dll)
        pltpu.sync_copy(irl_s.at[pl.ds(row0, NCH)], iv_srl)
        pltpu.sync_copy(irl_d.at[pl.ds(row0, NCH)], iv_drl)
        sems = (s0, s1, s2, s3)
        _gather_stream(hl, iv_sll, o_hs_ll, bufs, sems, wid)
        _gather_stream(hl, iv_dll, o_hd_ll, bufs, sems, wid)
        _gather_stream(hr, iv_srl, o_hs_rl, bufs, sems, wid)
        _gather_stream(hl, iv_drl, o_hd_rl, bufs, sems, wid)
        _gather_stream(xl, iv_sll, o_xs_ll, bufs, sems, wid)
        _gather_stream(xl, iv_dll, o_xl_ll, bufs, sems, wid)
        _gather_stream(xr, iv_srl, o_xs_rl, bufs, sems, wid)
        _gather_stream(xl, iv_drl, o_xl_rl, bufs, sems, wid)

    return k(hl_t, hr_t, xl_t, xr_t, src_ll2, dst_ll2, src_rl2, dst_rl2)


def _scatter_stream(msg, idx_v, acc, bufs, semA, semB, wid):
    def l_mk(g, slot, sem):
        off = pl.multiple_of((wid * NCH + g) * C, 8)
        return pltpu.make_async_copy(msg.at[pl.ds(off, C)], bufs.at[slot], sem)

    def l_issue(g, slot, sem):
        off = pl.multiple_of((wid * NCH + g) * C, 8)
        pltpu.async_copy(msg.at[pl.ds(off, C)], bufs.at[slot], sem)

    def scat(g, slot):
        pltpu.sync_copy(bufs.at[slot], acc.at[idx_v.at[g]], add=True)

    l_issue(0, 0, semA)
    l_issue(1, 1, semA)

    def body(i, carry):
        g0 = i * 4
        l_mk(g0, 0, semA).wait()
        l_mk(g0 + 1, 1, semA).wait()
        l_issue(g0 + 2, 2, semB)
        l_issue(g0 + 3, 3, semB)
        scat(g0, 0)
        scat(g0 + 1, 1)
        l_mk(g0 + 2, 2, semB).wait()
        l_mk(g0 + 3, 3, semB).wait()

        @pl.when(g0 + 4 < NCH)
        def _():
            l_issue(g0 + 4, 0, semA)
            l_issue(g0 + 5, 1, semA)

        scat(g0 + 2, 2)
        scat(g0 + 3, 3)
        return carry

    lax.fori_loop(0, NCH // 4, body, 0)


def _sc_scatter_h(mh_ll, mh_rl, dlo_ll2, dlo_rl2, dhi_ll2, dhi_rl2, zeros_c):
    mesh = plsc.VectorSubcoreMesh(core_axis_name="c", subcore_axis_name="s")

    @functools.partial(
        pl.kernel, mesh=mesh,
        out_type=jax.ShapeDtypeStruct((NC, 2 * HACC_N, 128), jnp.float32),
        scratch_types=[
            pltpu.VMEM((NCH, C), jnp.int32),
            pltpu.VMEM((NCH, C), jnp.int32),
            pltpu.VMEM((4, C, 128), jnp.float32),
            pltpu.VMEM_SHARED((HACC_N, 128), jnp.float32),
        ] + [pltpu.SemaphoreType.DMA] * 2,
    )
    def k(mhll, mhrl, lo_ll, lo_rl, hi_ll, hi_rl, zc, o_h,
          iv_ll, iv_rl, bufs, hacc, sA, sB):
        cid = lax.axis_index("c")
        sid = lax.axis_index("s")
        wid = sid * NC + cid
        tr0 = sid * HROWS_T
        row0 = wid * NCH
        pltpu.sync_copy(zc, hacc.at[pl.ds(tr0, HROWS_T)])
        pltpu.sync_copy(lo_ll.at[pl.ds(row0, NCH)], iv_ll)
        pltpu.sync_copy(lo_rl.at[pl.ds(row0, NCH)], iv_rl)
        plsc.subcore_barrier()
        _scatter_stream(mhll, iv_ll, hacc, bufs, sA, sB, wid)
        _scatter_stream(mhrl, iv_rl, hacc, bufs, sA, sB, wid)
        plsc.subcore_barrier()
        pltpu.sync_copy(hacc.at[pl.ds(tr0, HROWS_T)],
                        o_h.at[cid].at[pl.ds(tr0, HROWS_T)])
        pltpu.sync_copy(zc, hacc.at[pl.ds(tr0, HROWS_T)])
        pltpu.sync_copy(hi_ll.at[pl.ds(row0, NCH)], iv_ll)
        pltpu.sync_copy(hi_rl.at[pl.ds(row0, NCH)], iv_rl)
        plsc.subcore_barrier()
        _scatter_stream(mhll, iv_ll, hacc, bufs, sA, sB, wid)
        _scatter_stream(mhrl, iv_rl, hacc, bufs, sA, sB, wid)
        plsc.subcore_barrier()
        pltpu.sync_copy(hacc.at[pl.ds(tr0, HROWS_T)],
                        o_h.at[cid].at[pl.ds(HACC_N + tr0, HROWS_T)])

    return k(mh_ll, mh_rl, dlo_ll2, dlo_rl2, dhi_ll2, dhi_rl2, zeros_c)


def _sc_scatter_x(mx_ll, mx_rl, dst8_ll2, dst8_rl2, zeros_c):
    mesh = plsc.VectorSubcoreMesh(core_axis_name="c", subcore_axis_name="s")

    @functools.partial(
        pl.kernel, mesh=mesh,
        out_type=jax.ShapeDtypeStruct((NC, XACC_N, 128), jnp.float32),
        scratch_types=[
            pltpu.VMEM((NCH, C), jnp.int32),
            pltpu.VMEM((NCH, C), jnp.int32),
            pltpu.VMEM((4, C, 128), jnp.float32),
            pltpu.VMEM_SHARED((XACC_N, 128), jnp.float32),
        ] + [pltpu.SemaphoreType.DMA] * 2,
    )
    def k(mxll, mxrl, d8ll, d8rl, zc, o_x,
          iv8_ll, iv8_rl, bufs, xacc, sA, sB):
        cid = lax.axis_index("c")
        sid = lax.axis_index("s")
        wid = sid * NC + cid
        xtr0 = sid * XROWS_T
        row0 = wid * NCH
        pltpu.sync_copy(zc.at[pl.ds(0, XROWS_T)], xacc.at[pl.ds(xtr0, XROWS_T)])
        pltpu.sync_copy(d8ll.at[pl.ds(row0, NCH)], iv8_ll)
        pltpu.sync_copy(d8rl.at[pl.ds(row0, NCH)], iv8_rl)
        plsc.subcore_barrier()
        _scatter_stream(mxll, iv8_ll, xacc, bufs, sA, sB, wid)
        _scatter_stream(mxrl, iv8_rl, xacc, bufs, sA, sB, wid)
        plsc.subcore_barrier()
        pltpu.sync_copy(xacc.at[pl.ds(xtr0, XROWS_T)],
                        o_x.at[cid].at[pl.ds(xtr0, XROWS_T)])

    return k(mx_ll, mx_rl, dst8_ll2, dst8_rl2, zeros_c)


def _edge_body(hs_ref, hd_ref, xs_ref, xl_ref,
               w1a, w1b, w1c, b1, w2, b2,
               c1a, c1b, c1c, bc1, c2, bc2,
               msgh_ref, msgx_ref):
    hs = hs_ref[...]
    hd = hd_ref[...]
    xs = xs_ref[...]
    xl = xl_ref[...]
    lane = lax.broadcasted_iota(jnp.int32, (1, 128), 1)
    xd = jnp.where(lane < 3, xs - xl, 0.0)
    d2 = jnp.sum(xd * xd, axis=1, keepdims=True) + 1e-12
    dij = jnp.sqrt(d2)
    xdn = xd / (dij + 1e-9)
    pre = (jnp.dot(hs, w1a[...], preferred_element_type=jnp.float32)
           + jnp.dot(hd, w1b[...], preferred_element_type=jnp.float32)
           + dij * w1c[...] + b1[...])
    y = _silu(pre)
    msgh_ref[...] = _silu(jnp.dot(y, w2[...], preferred_element_type=jnp.float32)
                          + b2[...])
    prec = (jnp.dot(hs, c1a[...], preferred_element_type=jnp.float32)
            + jnp.dot(hd, c1b[...], preferred_element_type=jnp.float32)
            + dij * c1c[...] + bc1[...])
    yc = _silu(prec)
    scale = _silu(jnp.dot(yc, c2[...], preferred_element_type=jnp.float32)
                  + bc2[...])
    slot = xl[:, 3:4]
    place = (lane // 16).astype(jnp.float32) == slot
    xdn_t = jnp.concatenate([xdn[:, :16]] * 8, axis=1)
    msgx_ref[...] = jnp.where(place, scale * xdn_t, 0.0)


def _edge_compute(hs, hd, xs, xl, ew):
    e = hs.shape[0]
    grid = (e // _BE,)
    blk = pl.BlockSpec((_BE, 128), lambda i: (i, 0))
    full = lambda a: pl.BlockSpec(a.shape, lambda i: (0, 0))
    in_specs = [blk, blk, blk, blk] + [full(w) for w in ew]
    out_specs = [blk, blk]
    return pl.pallas_call(
        _edge_body,
        grid=grid,
        in_specs=in_specs,
        out_specs=out_specs,
        out_shape=[jax.ShapeDtypeStruct((e, 128), jnp.float32),
                   jax.ShapeDtypeStruct((e, 128), jnp.float32)],
    )(hs, hd, xs, xl, *ew)


def _node_body(h_ref, hn0_ref, hn1_ref, x_ref, xn_ref,
               n1a, n1b, nb1, n2, nb2, hout_ref, xout_ref):
    h = h_ref[...]
    hn = hn0_ref[...] + hn1_ref[...]
    y = _silu(jnp.dot(h, n1a[...], preferred_element_type=jnp.float32)
              + jnp.dot(hn, n1b[...], preferred_element_type=jnp.float32)
              + nb1[...])
    hout_ref[...] = h + jnp.dot(y, n2[...], preferred_element_type=jnp.float32) + nb2[...]
    xout_ref[...] = x_ref[...] + xn_ref[...]


def _node_compute(h, hn0, hn1, x16, xn16, nw):
    n = h.shape[0]
    grid = (n // _BN,)
    blk = pl.BlockSpec((_BN, 128), lambda i: (i, 0))
    blkx = pl.BlockSpec((_BN, 16), lambda i: (i, 0))
    full = lambda a: pl.BlockSpec(a.shape, lambda i: (0, 0))
    return pl.pallas_call(
        _node_body,
        grid=grid,
        in_specs=[blk, blk, blk, blkx, blkx] + [full(w) for w in nw],
        out_specs=[blk, blkx],
        out_shape=[jax.ShapeDtypeStruct((n, 128), jnp.float32),
                   jax.ShapeDtypeStruct((n, 16), jnp.float32)],
    )(h, hn0, hn1, x16, xn16, *nw)


def _split_edge_weights(lp, et, in_s):
    w1 = lp['edge_' + et + '_1']['W']
    b1 = lp['edge_' + et + '_1']['b']
    w2 = lp['edge_' + et + '_2']['W']
    b2 = lp['edge_' + et + '_2']['b']
    c1 = lp['coord_' + et + '_1']['W']
    bc1 = lp['coord_' + et + '_1']['b']
    c2 = lp['coord_' + et + '_2']['W']
    bc2 = lp['coord_' + et + '_2']['b']
    return (w1[:in_s], w1[in_s:2 * in_s], w1[2 * in_s:2 * in_s + 1], b1[None, :],
            w2, b2[None, :],
            c1[:in_s], c1[in_s:2 * in_s], c1[2 * in_s:2 * in_s + 1], bc1[None, :],
            c2, bc2[None, :])


def _pad_idx(a, fill):
    return jnp.concatenate(
        [a.astype(jnp.int32),
         jnp.full((E_PAD - N_EDGE,), fill, jnp.int32)]).reshape(NW * NCH, C)


def _x_table(x, n_rows):
    n = x.shape[0]
    mod8 = (jnp.arange(n_rows, dtype=jnp.int32) % 8).astype(jnp.float32)
    t = jnp.zeros((n_rows, 128), jnp.float32)
    t = t.at[:n, :3].set(x)
    t = t.at[:, 3].set(mod8)
    return t


def kernel(h_lig, h_rec, x_lig, x_rec, edge_index_ll, edge_index_rl, params):
    src_ll2 = _pad_idx(edge_index_ll[0], 0)
    dst_ll2 = _pad_idx(edge_index_ll[1], DPAD)
    src_rl2 = _pad_idx(edge_index_rl[0], 0)
    dst_rl2 = _pad_idx(edge_index_rl[1], DPAD)
    dst8_ll2 = dst_ll2 // 8
    dst8_rl2 = dst_rl2 // 8
    dlo_ll2 = jnp.where(dst_ll2 < HALF, dst_ll2, HALF)
    dlo_rl2 = jnp.where(dst_rl2 < HALF, dst_rl2, HALF)
    dhi_ll2 = jnp.where(dst_ll2 >= HALF, dst_ll2 - HALF, HALF)
    dhi_rl2 = jnp.where(dst_rl2 >= HALF, dst_rl2 - HALF, HALF)
    hr_t = jnp.pad(h_rec, ((0, N_TAB - N_NODE), (0, 0)))
    xr_t = _x_table(x_rec, N_TAB)
    zeros_c = jnp.zeros((HROWS_T, 128), jnp.float32)

    hl = h_lig
    xl3 = x_lig
    for lp in params:
        in_s = hl.shape[1]
        hl_t = jnp.pad(hl, ((0, N_TAB - N_NODE), (0, 0)))
        xl_t = _x_table(xl3, N_TAB)
        (hs_ll, hd_ll, xs_ll, xl_ll,
         hs_rl, hd_rl, xs_rl, xl_rl) = _sc_gather_layer(
            hl_t, hr_t, xl_t, xr_t, src_ll2, dst_ll2, src_rl2, dst_rl2)
        mh_ll, mx_ll = _edge_compute(hs_ll, hd_ll, xs_ll, xl_ll,
                                     _split_edge_weights(lp, 'll', in_s))
        mh_rl, mx_rl = _edge_compute(hs_rl, hd_rl, xs_rl, xl_rl,
                                     _split_edge_weights(lp, 'rl', in_s))
        h_parts = _sc_scatter_h(mh_ll, mh_rl, dlo_ll2, dlo_rl2,
                                dhi_ll2, dhi_rl2, zeros_c)
        x_parts = _sc_scatter_x(mx_ll, mx_rl, dst8_ll2, dst8_rl2, zeros_c)
        hn0 = jnp.concatenate([h_parts[0, :HALF],
                               h_parts[0, HACC_N:HACC_N + N_NODE - HALF]])
        hn1 = jnp.concatenate([h_parts[1, :HALF],
                               h_parts[1, HACC_N:HACC_N + N_NODE - HALF]])
        xn16 = (x_parts[0] + x_parts[1]).reshape(XACC_N * 8, 16)[:N_NODE]
        x16 = jnp.pad(xl3, ((0, 0), (0, 13)))
        n1 = lp['node_1']['W']
        nw = (n1[:in_s], n1[in_s:], lp['node_1']['b'][None, :],
              lp['node_2']['W'], lp['node_2']['b'][None, :])
        hl, x16o = _node_compute(hl, hn0, hn1, x16, xn16, nw)
        xl3 = x16o[:, :3]
    return (hl, h_rec, xl3, x_rec)
